# baseline (speedup 1.0000x reference)
import jax
import jax.numpy as jnp
from jax.experimental import pallas as pl
from jax.experimental.pallas import tpu as pltpu

_MLM_PROBABILITY = 0.15
_PAD_TOKEN = 0
_K = 8
_CB = 16


def _rtd_kernel(train_ref, pos_hbm, vec_ref, ids_ref, rand_ref,
                out_hbm, labels_ref, mask_ref,
                in_buf, out_buf, in_sems, out_sems):
    n_chunks = pos_hbm.shape[0] // _CB
    T = pos_hbm.shape[1]
    H = pos_hbm.shape[2]
    is_train = train_ref[0] != 0

    def in_copy(c, s):
        return pltpu.make_async_copy(
            pos_hbm.at[pl.ds(c * _CB, _CB)], in_buf.at[s], in_sems.at[s])

    def out_copy(c, s):
        return pltpu.make_async_copy(
            out_buf.at[s], out_hbm.at[pl.ds(c * _CB, _CB)], out_sems.at[s])

    for s in range(_K):
        in_copy(s, s).start()

    def body(c, carry):
        s = jax.lax.rem(c, _K)
        in_copy(c, s).wait()

        r0 = c * _CB
        ids = ids_ref[pl.ds(r0, _CB), :]
        non_pad = (ids != _PAD_TOKEN).astype(jnp.int32)
        train_m = rand_ref[pl.ds(r0, _CB), :] * non_pad

        last = (jnp.sum(non_pad, axis=1) - 1)[:, None]
        col = jax.lax.broadcasted_iota(jnp.int32, ids.shape, 1)
        eval_m = jnp.where(col == last, non_pad, 0)

        m = jnp.where(is_train, train_m, eval_m)

        mask_ref[pl.ds(r0, _CB), :] = m != 0
        labels_ref[pl.ds(r0, _CB), :] = m * ids

        @pl.when(c >= _K)
        def _():
            out_copy(c - _K, s).wait()

        m3 = m[:, :, None]
        vec = vec_ref[...].reshape(1, 1, H)
        out_buf[s] = jnp.where(m3 != 0, vec, in_buf[s])

        out_copy(c, s).start()

        @pl.when(c + _K < n_chunks)
        def _():
            in_copy(c + _K, s).start()

        return carry

    jax.lax.fori_loop(0, n_chunks, body, 0)

    for i in range(_K):
        c = n_chunks - _K + i
        out_copy(c, c % _K).wait()


def kernel(pos_emb, masked_item_embedding, itemid_seq, training):
    B, T, H = pos_emb.shape
    ids = itemid_seq.astype(jnp.int32)
    probs = jax.random.uniform(jax.random.key(42), (B, T), dtype=jnp.float32)
    rand_mask = (probs < _MLM_PROBABILITY).astype(jnp.int32)
    train_flag = jnp.asarray(training, jnp.int32).reshape(1)
    vec = masked_item_embedding.astype(pos_emb.dtype).reshape(1, H)

    out_shapes = (
        jax.ShapeDtypeStruct((B, T, H), pos_emb.dtype),
        jax.ShapeDtypeStruct((B, T), ids.dtype),
        jax.ShapeDtypeStruct((B, T), jnp.bool_),
    )
    vmem = pl.BlockSpec(memory_space=pltpu.VMEM)
    pos_out, labels, mask_labels = pl.pallas_call(
        _rtd_kernel,
        in_specs=[
            pl.BlockSpec(memory_space=pltpu.SMEM),
            pl.BlockSpec(memory_space=pltpu.MemorySpace.HBM),
            vmem, vmem, vmem,
        ],
        out_specs=(
            pl.BlockSpec(memory_space=pltpu.MemorySpace.HBM),
            vmem, vmem,
        ),
        out_shape=out_shapes,
        scratch_shapes=[
            pltpu.VMEM((_K, _CB, T, H), jnp.float32),
            pltpu.VMEM((_K, _CB, T, H), jnp.float32),
            pltpu.SemaphoreType.DMA((_K,)),
            pltpu.SemaphoreType.DMA((_K,)),
        ],
    )(train_flag, pos_emb, vec, ids, rand_mask)

    return (pos_out, labels, mask_labels)


# speedup vs baseline: 1.1977x; 1.1977x over previous
import jax
import jax.numpy as jnp
from jax.experimental import pallas as pl
from jax.experimental.pallas import tpu as pltpu

_MLM_PROBABILITY = 0.15
_PAD_TOKEN = 0
_B_BLK = 64


def _rtd_kernel(train_ref, pos_ref, vec2_ref, ids_e_ref, ids_o_ref,
                rand_e_ref, rand_o_ref,
                out_ref, lab_e_ref, lab_o_ref, mask_e_ref, mask_o_ref):
    ids_e = ids_e_ref[...]
    ids_o = ids_o_ref[...]
    np_e = (ids_e != _PAD_TOKEN).astype(jnp.int32)
    np_o = (ids_o != _PAD_TOKEN).astype(jnp.int32)
    train_e = rand_e_ref[...] * np_e
    train_o = rand_o_ref[...] * np_o

    cnt = jnp.sum(np_e, axis=1) + jnp.sum(np_o, axis=1)
    last = (cnt - 1)[:, None]
    p2 = 2 * jax.lax.broadcasted_iota(jnp.int32, ids_e.shape, 1)
    eval_e = jnp.where(p2 == last, np_e, 0)
    eval_o = jnp.where(p2 + 1 == last, np_o, 0)

    is_train = train_ref[0] != 0
    m_e = jnp.where(is_train, train_e, eval_e)
    m_o = jnp.where(is_train, train_o, eval_o)

    mask_e_ref[...] = m_e != 0
    mask_o_ref[...] = m_o != 0
    lab_e_ref[...] = m_e * ids_e
    lab_o_ref[...] = m_o * ids_o

    B, P = m_e.shape
    me3 = jnp.broadcast_to(m_e[:, :, None], (B, P, 128))
    mo3 = jnp.broadcast_to(m_o[:, :, None], (B, P, 128))
    lane = jax.lax.broadcasted_iota(jnp.int32, (B, P, 128), 2)
    mexp = jnp.where(lane < 64, me3, mo3)
    vec2 = vec2_ref[...].reshape(1, 1, 128)
    out_ref[...] = jnp.where(mexp != 0, vec2, pos_ref[...])


def kernel(pos_emb, masked_item_embedding, itemid_seq, training):
    B, T, H = pos_emb.shape
    P = T // 2
    ids = itemid_seq.astype(jnp.int32)
    probs = jax.random.uniform(jax.random.key(42), (B, T), dtype=jnp.float32)
    rand_mask = (probs < _MLM_PROBABILITY).astype(jnp.int32)
    train_flag = jnp.asarray(training, jnp.int32).reshape(1)
    vec = masked_item_embedding.astype(pos_emb.dtype)
    vec2 = jnp.concatenate([vec, vec]).reshape(1, 2 * H)

    pos2 = pos_emb.reshape(B, P, 2 * H)
    ids_e, ids_o = ids[:, 0::2], ids[:, 1::2]
    rand_e, rand_o = rand_mask[:, 0::2], rand_mask[:, 1::2]

    grid = (B // _B_BLK,)
    out_shapes = (
        jax.ShapeDtypeStruct((B, P, 2 * H), pos_emb.dtype),
        jax.ShapeDtypeStruct((B, P), ids.dtype),
        jax.ShapeDtypeStruct((B, P), ids.dtype),
        jax.ShapeDtypeStruct((B, P), jnp.bool_),
        jax.ShapeDtypeStruct((B, P), jnp.bool_),
    )
    small = pl.BlockSpec((_B_BLK, P), lambda i: (i, 0))
    big = pl.BlockSpec((_B_BLK, P, 2 * H), lambda i: (i, 0, 0))
    pos_out, lab_e, lab_o, m_e, m_o = pl.pallas_call(
        _rtd_kernel,
        grid=grid,
        in_specs=[
            pl.BlockSpec(memory_space=pltpu.SMEM),
            big,
            pl.BlockSpec((1, 2 * H), lambda i: (0, 0)),
            small, small, small, small,
        ],
        out_specs=(big, small, small, small, small),
        out_shape=out_shapes,
        compiler_params=pltpu.CompilerParams(
            dimension_semantics=("arbitrary",),
        ),
    )(train_flag, pos2, vec2, ids_e, ids_o, rand_e, rand_o)

    pos_emb_inp = pos_out.reshape(B, T, H)
    labels = jnp.stack([lab_e, lab_o], axis=2).reshape(B, T)
    mask_labels = jnp.stack([m_e, m_o], axis=2).reshape(B, T)
    return (pos_emb_inp, labels, mask_labels)


# speedup vs baseline: 1.4967x; 1.2497x over previous
import jax
import jax.numpy as jnp
from jax.experimental import pallas as pl
from jax.experimental.pallas import tpu as pltpu

_MLM_PROBABILITY = 0.15
_PAD_TOKEN = 0
_K = 8
_CB = 32


def _rtd_kernel(train_ref, pos_hbm, vect_ref, ids_ref, rand_ref,
                out_hbm, labels_ref, mask_ref,
                in_buf, out_buf, in_sems, out_sems):
    n_chunks = pos_hbm.shape[0] // _CB
    W = pos_hbm.shape[1]
    T = ids_ref.shape[1]
    is_train = train_ref[0] != 0

    def in_copy(c, s):
        return pltpu.make_async_copy(
            pos_hbm.at[pl.ds(c * _CB, _CB)], in_buf.at[s], in_sems.at[s])

    def out_copy(c, s):
        return pltpu.make_async_copy(
            out_buf.at[s], out_hbm.at[pl.ds(c * _CB, _CB)], out_sems.at[s])

    for s in range(_K):
        in_copy(s, s).start()

    def body(c, carry):
        s = jax.lax.rem(c, _K)
        in_copy(c, s).wait()

        r0 = c * _CB
        ids = ids_ref[pl.ds(r0, _CB), :]
        non_pad = (ids != _PAD_TOKEN).astype(jnp.int32)
        train_m = rand_ref[pl.ds(r0, _CB), :] * non_pad

        last = (jnp.sum(non_pad, axis=1) - 1)[:, None]
        col = jax.lax.broadcasted_iota(jnp.int32, ids.shape, 1)
        eval_m = jnp.where(col == last, non_pad, 0)

        m = jnp.where(is_train, train_m, eval_m)

        mask_ref[pl.ds(r0, _CB), :] = m != 0
        labels_ref[pl.ds(r0, _CB), :] = m * ids

        @pl.when(c >= _K)
        def _():
            out_copy(c - _K, s).wait()

        half = jax.lax.broadcasted_iota(jnp.int32, (_CB, 128), 1) < 64
        pieces = []
        for k in range(T // 2):
            a = jnp.broadcast_to(m[:, 2 * k:2 * k + 1], (_CB, 128))
            b = jnp.broadcast_to(m[:, 2 * k + 1:2 * k + 2], (_CB, 128))
            pieces.append(jnp.where(half, a, b))
        mexp = jnp.concatenate(pieces, axis=1)

        vect = vect_ref[...]
        out_buf[s] = jnp.where(mexp != 0, vect, in_buf[s])

        out_copy(c, s).start()

        @pl.when(c + _K < n_chunks)
        def _():
            in_copy(c + _K, s).start()

        return carry

    jax.lax.fori_loop(0, n_chunks, body, 0)

    for i in range(_K):
        c = n_chunks - _K + i
        out_copy(c, c % _K).wait()


def kernel(pos_emb, masked_item_embedding, itemid_seq, training):
    B, T, H = pos_emb.shape
    W = T * H
    ids = itemid_seq.astype(jnp.int32)
    probs = jax.random.uniform(jax.random.key(42), (B, T), dtype=jnp.float32)
    rand_mask = (probs < _MLM_PROBABILITY).astype(jnp.int32)
    train_flag = jnp.asarray(training, jnp.int32).reshape(1)
    vec = masked_item_embedding.astype(pos_emb.dtype)
    vec_tiled = jnp.tile(vec, T).reshape(1, W)

    pos2d = pos_emb.reshape(B, W)

    out_shapes = (
        jax.ShapeDtypeStruct((B, W), pos_emb.dtype),
        jax.ShapeDtypeStruct((B, T), ids.dtype),
        jax.ShapeDtypeStruct((B, T), jnp.bool_),
    )
    vmem = pl.BlockSpec(memory_space=pltpu.VMEM)
    pos_out, labels, mask_labels = pl.pallas_call(
        _rtd_kernel,
        in_specs=[
            pl.BlockSpec(memory_space=pltpu.SMEM),
            pl.BlockSpec(memory_space=pltpu.MemorySpace.HBM),
            vmem, vmem, vmem,
        ],
        out_specs=(
            pl.BlockSpec(memory_space=pltpu.MemorySpace.HBM),
            vmem, vmem,
        ),
        out_shape=out_shapes,
        scratch_shapes=[
            pltpu.VMEM((_K, _CB, W), jnp.float32),
            pltpu.VMEM((_K, _CB, W), jnp.float32),
            pltpu.SemaphoreType.DMA((_K,)),
            pltpu.SemaphoreType.DMA((_K,)),
        ],
    )(train_flag, pos2d, vec_tiled, ids, rand_mask)

    return (pos_out.reshape(B, T, H), labels, mask_labels)


# speedup vs baseline: 5.6530x; 3.7770x over previous
import jax
import jax.numpy as jnp
from jax.experimental import pallas as pl
from jax.experimental.pallas import tpu as pltpu

_MLM_PROBABILITY = 0.15
_PAD_TOKEN = 0
_K = 8
_CT = 2


def _rtd_kernel(train_ref, pos_hbm, vec_ref, ids_ref, rand_ref,
                out_hbm, labels_ref, mask_ref,
                m_buf, in_buf, out_buf, in_sems, out_sems):
    T, H, B = pos_hbm.shape
    n_chunks = T // _CT
    is_train = train_ref[0] != 0

    def in_copy(c, s):
        return pltpu.make_async_copy(
            pos_hbm.at[pl.ds(c * _CT, _CT)], in_buf.at[s], in_sems.at[s])

    def out_copy(c, s):
        return pltpu.make_async_copy(
            out_buf.at[s], out_hbm.at[pl.ds(c * _CT, _CT)], out_sems.at[s])

    for s in range(_K):
        in_copy(s, s).start()

    ids = ids_ref[...]
    non_pad = (ids != _PAD_TOKEN).astype(jnp.int32)
    train_m = rand_ref[...] * non_pad

    last = (jnp.sum(non_pad, axis=0) - 1)[None, :]
    row = jax.lax.broadcasted_iota(jnp.int32, ids.shape, 0)
    eval_m = jnp.where(row == last, non_pad, 0)

    m = jnp.where(is_train, train_m, eval_m)
    mask_ref[...] = m != 0
    labels_ref[...] = m * ids
    for k in range(n_chunks):
        m_buf[k] = m[k * _CT:(k + 1) * _CT, :]

    vec3 = jax.lax.broadcast_in_dim(vec_ref[...], (_CT, H, B), (1, 2))

    def body(c, carry):
        s = jax.lax.rem(c, _K)
        in_copy(c, s).wait()

        @pl.when(c >= _K)
        def _():
            out_copy(c - _K, s).wait()

        mc = m_buf[c]
        m3 = jax.lax.broadcast_in_dim(mc, (_CT, H, B), (0, 2))
        out_buf[s] = jnp.where(m3 != 0, vec3, in_buf[s])

        out_copy(c, s).start()

        @pl.when(c + _K < n_chunks)
        def _():
            in_copy(c + _K, s).start()

        return carry

    jax.lax.fori_loop(0, n_chunks, body, 0)

    for i in range(_K):
        c = n_chunks - _K + i
        out_copy(c, c % _K).wait()


def kernel(pos_emb, masked_item_embedding, itemid_seq, training):
    B, T, H = pos_emb.shape
    ids = itemid_seq.astype(jnp.int32)
    probs = jax.random.uniform(jax.random.key(42), (B, T), dtype=jnp.float32)
    rand_mask = (probs < _MLM_PROBABILITY).astype(jnp.int32)
    train_flag = jnp.asarray(training, jnp.int32).reshape(1)
    vec = masked_item_embedding.astype(pos_emb.dtype).reshape(H, 1)

    pos_t = jnp.transpose(pos_emb, (1, 2, 0))
    ids_t = ids.T
    rand_t = rand_mask.T

    out_shapes = (
        jax.ShapeDtypeStruct((T, H, B), pos_emb.dtype),
        jax.ShapeDtypeStruct((T, B), ids.dtype),
        jax.ShapeDtypeStruct((T, B), jnp.bool_),
    )
    vmem = pl.BlockSpec(memory_space=pltpu.VMEM)
    out_t, labels_t, mask_t = pl.pallas_call(
        _rtd_kernel,
        in_specs=[
            pl.BlockSpec(memory_space=pltpu.SMEM),
            pl.BlockSpec(memory_space=pltpu.MemorySpace.HBM),
            vmem, vmem, vmem,
        ],
        out_specs=(
            pl.BlockSpec(memory_space=pltpu.MemorySpace.HBM),
            vmem, vmem,
        ),
        out_shape=out_shapes,
        compiler_params=pltpu.CompilerParams(
            vmem_limit_bytes=60 * 1024 * 1024,
        ),
        scratch_shapes=[
            pltpu.VMEM((T // _CT, _CT, B), jnp.int32),
            pltpu.VMEM((_K, _CT, H, B), jnp.float32),
            pltpu.VMEM((_K, _CT, H, B), jnp.float32),
            pltpu.SemaphoreType.DMA((_K,)),
            pltpu.SemaphoreType.DMA((_K,)),
        ],
    )(train_flag, pos_t, vec, ids_t, rand_t)

    pos_emb_inp = jnp.transpose(out_t, (2, 0, 1))
    return (pos_emb_inp, labels_t.T, mask_t.T)


# speedup vs baseline: 6.2191x; 1.1001x over previous
import jax
import jax.numpy as jnp
from jax.experimental import pallas as pl
from jax.experimental.pallas import tpu as pltpu

_MLM_PROBABILITY = 0.15
_PAD_TOKEN = 0
_K = 8
_CT = 2


def _rtd_kernel(train_ref, pos_hbm, vec_ref, ids_ref, rand_ref,
                out_hbm, labels_ref, mask_ref,
                m_buf, in_buf, out_buf, in_sems, out_sems):
    T, H, B = pos_hbm.shape
    n_chunks = T // _CT
    is_train = train_ref[0] != 0

    def in_copy(c, s):
        return pltpu.make_async_copy(
            pos_hbm.at[pl.ds(c * _CT, _CT)], in_buf.at[s], in_sems.at[s])

    def out_copy(c, s):
        return pltpu.make_async_copy(
            out_buf.at[s], out_hbm.at[pl.ds(c * _CT, _CT)], out_sems.at[s])

    for s in range(_K):
        in_copy(s, s).start()

    ids = ids_ref[...]
    non_pad = (ids != _PAD_TOKEN).astype(jnp.int32)
    train_m = rand_ref[...] * non_pad

    last = (jnp.sum(non_pad, axis=0) - 1)[None, :]
    row = jax.lax.broadcasted_iota(jnp.int32, ids.shape, 0)
    eval_m = jnp.where(row == last, non_pad, 0)

    m = jnp.where(is_train, train_m, eval_m)
    mask_ref[...] = m != 0
    labels_ref[...] = m * ids
    for k in range(n_chunks):
        m_buf[k] = m[k * _CT:(k + 1) * _CT, :]

    vec3 = jax.lax.broadcast_in_dim(vec_ref[...], (_CT, H, B), (1, 2))

    def body(c, carry):
        s = jax.lax.rem(c, _K)
        in_copy(c, s).wait()

        @pl.when(c >= _K)
        def _():
            out_copy(c - _K, s).wait()

        mc = m_buf[c]
        m3 = jax.lax.broadcast_in_dim(mc, (_CT, H, B), (0, 2))
        out_buf[s] = jnp.where(m3 != 0, vec3, in_buf[s])

        out_copy(c, s).start()

        @pl.when(c + _K < n_chunks)
        def _():
            in_copy(c + _K, s).start()

        return carry

    jax.lax.fori_loop(0, n_chunks, body, 0)

    for i in range(_K):
        c = n_chunks - _K + i
        out_copy(c, c % _K).wait()


def kernel(pos_emb, masked_item_embedding, itemid_seq, training):
    B, T, H = pos_emb.shape
    ids = itemid_seq.astype(jnp.int32)
    with jax.ensure_compile_time_eval():
        probs = jax.random.uniform(jax.random.key(42), (B, T), dtype=jnp.float32)
        rand_mask = (probs < _MLM_PROBABILITY).astype(jnp.int32)
        rand_t_const = rand_mask.T
    train_flag = jnp.asarray(training, jnp.int32).reshape(1)
    vec = masked_item_embedding.astype(pos_emb.dtype).reshape(H, 1)

    pos_t = jnp.transpose(pos_emb, (1, 2, 0))
    ids_t = ids.T
    rand_t = rand_t_const

    out_shapes = (
        jax.ShapeDtypeStruct((T, H, B), pos_emb.dtype),
        jax.ShapeDtypeStruct((T, B), ids.dtype),
        jax.ShapeDtypeStruct((T, B), jnp.bool_),
    )
    vmem = pl.BlockSpec(memory_space=pltpu.VMEM)
    out_t, labels_t, mask_t = pl.pallas_call(
        _rtd_kernel,
        in_specs=[
            pl.BlockSpec(memory_space=pltpu.SMEM),
            pl.BlockSpec(memory_space=pltpu.MemorySpace.HBM),
            vmem, vmem, vmem,
        ],
        out_specs=(
            pl.BlockSpec(memory_space=pltpu.MemorySpace.HBM),
            vmem, vmem,
        ),
        out_shape=out_shapes,
        compiler_params=pltpu.CompilerParams(
            vmem_limit_bytes=60 * 1024 * 1024,
        ),
        scratch_shapes=[
            pltpu.VMEM((T // _CT, _CT, B), jnp.int32),
            pltpu.VMEM((_K, _CT, H, B), jnp.float32),
            pltpu.VMEM((_K, _CT, H, B), jnp.float32),
            pltpu.SemaphoreType.DMA((_K,)),
            pltpu.SemaphoreType.DMA((_K,)),
        ],
    )(train_flag, pos_t, vec, ids_t, rand_t)

    pos_emb_inp = jnp.transpose(out_t, (2, 0, 1))
    return (pos_emb_inp, labels_t.T, mask_t.T)


# speedup vs baseline: 6.2460x; 1.0043x over previous
import jax
import jax.numpy as jnp
from jax.experimental import pallas as pl
from jax.experimental.pallas import tpu as pltpu

_MLM_PROBABILITY = 0.15
_PAD_TOKEN = 0
_K = 5
_CT = 4


def _rtd_kernel(train_ref, pos_hbm, vec_ref, ids_ref, rand_ref,
                out_hbm, labels_ref, mask_ref,
                m_buf, in_buf, out_buf, in_sems, out_sems):
    T, H, B = pos_hbm.shape
    n_chunks = T // _CT
    is_train = train_ref[0] != 0

    def in_copy(c, s):
        return pltpu.make_async_copy(
            pos_hbm.at[pl.ds(c * _CT, _CT)], in_buf.at[s], in_sems.at[s])

    def out_copy(c, s):
        return pltpu.make_async_copy(
            out_buf.at[s], out_hbm.at[pl.ds(c * _CT, _CT)], out_sems.at[s])

    for s in range(_K):
        in_copy(s, s).start()

    ids = ids_ref[...]
    non_pad = (ids != _PAD_TOKEN).astype(jnp.int32)
    train_m = rand_ref[...].astype(jnp.int32) * non_pad

    last = (jnp.sum(non_pad, axis=0) - 1)[None, :]
    row = jax.lax.broadcasted_iota(jnp.int32, ids.shape, 0)
    eval_m = jnp.where(row == last, non_pad, 0)

    m = jnp.where(is_train, train_m, eval_m)
    mask_ref[...] = m != 0
    labels_ref[...] = m * ids
    for k in range(n_chunks):
        m_buf[k] = m[k * _CT:(k + 1) * _CT, :]

    vec3 = jax.lax.broadcast_in_dim(vec_ref[...], (_CT, H, B), (1, 2))

    def body(c, carry):
        s = jax.lax.rem(c, _K)
        in_copy(c, s).wait()

        @pl.when(c >= _K)
        def _():
            out_copy(c - _K, s).wait()

        mc = m_buf[c]
        m3 = jax.lax.broadcast_in_dim(mc, (_CT, H, B), (0, 2))
        out_buf[s] = jnp.where(m3 != 0, vec3, in_buf[s])

        out_copy(c, s).start()

        @pl.when(c + _K < n_chunks)
        def _():
            in_copy(c + _K, s).start()

        return carry

    jax.lax.fori_loop(0, n_chunks, body, 0)

    for i in range(_K):
        c = n_chunks - _K + i
        out_copy(c, c % _K).wait()


def kernel(pos_emb, masked_item_embedding, itemid_seq, training):
    B, T, H = pos_emb.shape
    ids = itemid_seq.astype(jnp.int32)
    with jax.ensure_compile_time_eval():
        probs = jax.random.uniform(jax.random.key(42), (B, T), dtype=jnp.float32)
        rand_mask = (probs < _MLM_PROBABILITY).astype(jnp.int8)
        rand_t_const = rand_mask.T
    train_flag = jnp.asarray(training, jnp.int32).reshape(1)
    vec = masked_item_embedding.astype(pos_emb.dtype).reshape(H, 1)

    pos_t = jnp.transpose(pos_emb, (1, 2, 0))
    ids_t = ids.T
    rand_t = rand_t_const

    out_shapes = (
        jax.ShapeDtypeStruct((T, H, B), pos_emb.dtype),
        jax.ShapeDtypeStruct((T, B), ids.dtype),
        jax.ShapeDtypeStruct((T, B), jnp.bool_),
    )
    vmem = pl.BlockSpec(memory_space=pltpu.VMEM)
    out_t, labels_t, mask_t = pl.pallas_call(
        _rtd_kernel,
        in_specs=[
            pl.BlockSpec(memory_space=pltpu.SMEM),
            pl.BlockSpec(memory_space=pltpu.MemorySpace.HBM),
            vmem, vmem, vmem,
        ],
        out_specs=(
            pl.BlockSpec(memory_space=pltpu.MemorySpace.HBM),
            vmem, vmem,
        ),
        out_shape=out_shapes,
        compiler_params=pltpu.CompilerParams(
            vmem_limit_bytes=60 * 1024 * 1024,
        ),
        scratch_shapes=[
            pltpu.VMEM((T // _CT, _CT, B), jnp.int32),
            pltpu.VMEM((_K, _CT, H, B), jnp.float32),
            pltpu.VMEM((_K, _CT, H, B), jnp.float32),
            pltpu.SemaphoreType.DMA((_K,)),
            pltpu.SemaphoreType.DMA((_K,)),
        ],
    )(train_flag, pos_t, vec, ids_t, rand_t)

    pos_emb_inp = jnp.transpose(out_t, (2, 0, 1))
    return (pos_emb_inp, labels_t.T, mask_t.T)
